# baseline (speedup 1.0000x reference)
import functools

import jax
import jax.numpy as jnp
from jax import lax
from jax.experimental import pallas as pl
from jax.experimental.pallas import tpu as pltpu
from jax.experimental.pallas import tpu_sc as plsc

VOCAB = 100000
NUM_SEG = 2
MAX_LEN = 200
EMBED = 128
BATCH = 1024

NC, NS = 2, 16
NW = NC * NS
N = BATCH * MAX_LEN
CHUNK = 128
ROWS_PER_W = N // NW
CHUNKS_PER_W = ROWS_PER_W // CHUNK
LANES = 16
NGRP = 3
GRP = 2
PAIRS = CHUNKS_PER_W // GRP
PIECE = 40


def _sc_embed(seq3d, seg3d, token_table, segmt_table, pos_table):
    mesh = plsc.VectorSubcoreMesh(
        core_axis_name="c", subcore_axis_name="s",
        num_cores=NC, num_subcores=NS,
    )

    @functools.partial(
        pl.kernel,
        out_type=jax.ShapeDtypeStruct((N, EMBED), jnp.float32),
        mesh=mesh,
        scratch_types=(
            [pltpu.VMEM((CHUNKS_PER_W, CHUNK), jnp.int32)] * 2
            + [pltpu.VMEM((NGRP * GRP * CHUNK, EMBED), jnp.float32)]
            + [pltpu.VMEM((PIECE, EMBED), jnp.float32)]
            + [pltpu.VMEM((NUM_SEG, EMBED), jnp.float32)]
            + [pltpu.VMEM_SHARED((NUM_SEG * MAX_LEN, EMBED), jnp.float32)]
            + [pltpu.SemaphoreType.DMA] * (3 * NGRP)
        ),
    )
    def k(seq_hbm, seg_hbm, tok_hbm, st_hbm, pos_hbm, out_hbm,
          idx_tok, idx_cmb, *rest):
        rows_big = rest[0]
        tmp = rest[1]
        segv = rest[2]
        comb_sp = rest[3]
        sg = rest[4:4 + NGRP]
        sa = rest[4 + NGRP:4 + 2 * NGRP]
        so = rest[4 + 2 * NGRP:]

        sid = lax.axis_index("s")
        wid = sid * NC + lax.axis_index("c")
        obase = pl.multiple_of(wid * ROWS_PER_W, CHUNK)

        pltpu.sync_copy(seq_hbm.at[wid], idx_tok)

        def slot(g, q):
            return rows_big.at[pl.ds((g * GRP + q) * CHUNK, CHUNK)]

        def issue_tok_pair(p, g):
            for q in range(GRP):
                pltpu.async_copy(tok_hbm.at[idx_tok.at[p * GRP + q]],
                                 slot(g, q), sg[g])

        def wait_tok(p, g, q):
            pltpu.make_async_copy(tok_hbm.at[idx_tok.at[p * GRP + q]],
                                  slot(g, q), sg[g]).wait()

        def issue_add(p, g, q):
            pltpu.async_copy(comb_sp.at[idx_cmb.at[p * GRP + q]],
                             slot(g, q), sa[g], add=True)

        def wait_add(p, g, q):
            pltpu.make_async_copy(comb_sp.at[idx_cmb.at[p * GRP + q]],
                                  slot(g, q), sa[g]).wait()

        def store_src(g):
            return rows_big.at[pl.ds(g * GRP * CHUNK, GRP * CHUNK)]

        def out_at_pair(p):
            off = pl.multiple_of(obase + p * GRP * CHUNK, CHUNK)
            return out_hbm.at[pl.ds(off, GRP * CHUNK)]

        def issue_store(p, g):
            pltpu.async_copy(store_src(g), out_at_pair(p), so[g])

        def wait_store(p, g):
            pltpu.make_async_copy(store_src(g), out_at_pair(p), so[g]).wait()

        issue_tok_pair(0, 0)
        issue_tok_pair(1, 1)

        pltpu.sync_copy(seg_hbm.at[wid], idx_cmb)

        @pl.when(sid < MAX_LEN * NUM_SEG // PIECE)
        def _():
            pltpu.sync_copy(st_hbm, segv)
            poff = pl.multiple_of(lax.rem(sid, MAX_LEN // PIECE) * PIECE, 8)
            s2 = sid // (MAX_LEN // PIECE)
            pltpu.sync_copy(pos_hbm.at[pl.ds(poff, PIECE)], tmp)

            def add_body(r, c):
                for g in range(EMBED // LANES):
                    sl = pl.ds(g * LANES, LANES)
                    tmp[r, sl] = tmp[r, sl] + segv[s2, sl]
                return c

            lax.fori_loop(0, PIECE, add_body, 0)
            coff = pl.multiple_of(sid * PIECE, 8)
            pltpu.sync_copy(tmp, comb_sp.at[pl.ds(coff, PIECE)])

        def cidx_body(r, p):
            for g in range(EMBED // LANES):
                sl = pl.ds(g * LANES, LANES)
                idx_cmb[r, sl] = idx_cmb[r, sl] * MAX_LEN + p
                p = p + LANES
                p = jnp.where(p >= MAX_LEN, p - MAX_LEN, p)
            return p

        lax.fori_loop(0, CHUNKS_PER_W, cidx_body,
                      lax.broadcasted_iota(jnp.int32, (LANES,), 0))
        plsc.subcore_barrier()

        def phase(p, g, wait_prev_store=True, issue_next=True):
            for q in range(GRP):
                wait_tok(p, g, q)
                issue_add(p, g, q)
            for q in range(GRP):
                wait_add(p, g, q)
            issue_store(p, g)
            if wait_prev_store:
                wait_store(p - 1, (g + 2) % NGRP)
            if issue_next:
                issue_tok_pair(p + 2, (g + 2) % NGRP)

        phase(0, 0, wait_prev_store=False)
        phase(1, 1)

        def body(i, carry):
            p0 = 2 + i * NGRP
            for t in range(NGRP):
                phase(p0 + t, (2 + t) % NGRP)
            return carry

        lax.fori_loop(0, (PAIRS - 2 - GRP) // NGRP, body, 0)
        for p in range(PAIRS - GRP, PAIRS):
            phase(p, p % NGRP, wait_prev_store=False, issue_next=False)
        for p in range(PAIRS - NGRP, PAIRS):
            wait_store(p, p % NGRP)

    return k(seq3d, seg3d, token_table, segmt_table, pos_table)


def kernel(sequence, segment, token_table, segmt_table, pos_table):
    seq = sequence.astype(jnp.int32).reshape(NW, CHUNKS_PER_W, CHUNK)
    seg = segment.astype(jnp.int32).reshape(NW, CHUNKS_PER_W, CHUNK)
    out = _sc_embed(seq, seg, token_table, segmt_table, pos_table)
    return out.reshape(BATCH, MAX_LEN, EMBED)


# speedup vs baseline: 1.0078x; 1.0078x over previous
import functools

import jax
import jax.numpy as jnp
from jax import lax
from jax.experimental import pallas as pl
from jax.experimental.pallas import tpu as pltpu
from jax.experimental.pallas import tpu_sc as plsc

VOCAB = 100000
NUM_SEG = 2
MAX_LEN = 200
EMBED = 128
BATCH = 1024

NC, NS = 2, 16
NW = NC * NS
N = BATCH * MAX_LEN
CHUNK = 128
ROWS_PER_W = N // NW
CHUNKS_PER_W = ROWS_PER_W // CHUNK
LANES = 16
NBUF = 4
LOOKAHEAD = NBUF - 2
PIECE = 40


def _sc_embed(seq3d, seg3d, token_table, segmt_table, pos_table):
    mesh = plsc.VectorSubcoreMesh(
        core_axis_name="c", subcore_axis_name="s",
        num_cores=NC, num_subcores=NS,
    )

    @functools.partial(
        pl.kernel,
        out_type=jax.ShapeDtypeStruct((N, EMBED), jnp.float32),
        mesh=mesh,
        scratch_types=(
            [pltpu.VMEM((CHUNKS_PER_W, CHUNK), jnp.int32)] * 2
            + [pltpu.VMEM((CHUNK, EMBED), jnp.float32)] * NBUF
            + [pltpu.VMEM((NUM_SEG, EMBED), jnp.float32)]
            + [pltpu.VMEM_SHARED((NUM_SEG * MAX_LEN, EMBED), jnp.float32)]
            + [pltpu.SemaphoreType.DMA] * (3 * NBUF)
        ),
    )
    def k(seq_hbm, seg_hbm, tok_hbm, st_hbm, pos_hbm, out_hbm,
          idx_tok, idx_cmb, *rest):
        rows = rest[:NBUF]
        segv = rest[NBUF]
        comb_sp = rest[NBUF + 1]
        sg = rest[NBUF + 2:NBUF + 2 + NBUF]
        sa = rest[NBUF + 2 + NBUF:NBUF + 2 + 2 * NBUF]
        so = rest[NBUF + 2 + 2 * NBUF:]

        sid = lax.axis_index("s")
        wid = sid * NC + lax.axis_index("c")
        obase = pl.multiple_of(wid * ROWS_PER_W, CHUNK)

        pltpu.sync_copy(seq_hbm.at[wid], idx_tok)

        def out_at(j):
            return out_hbm.at[pl.ds(pl.multiple_of(obase + j * CHUNK, CHUNK),
                                    CHUNK)]

        def issue_tok(j, b):
            pltpu.async_copy(tok_hbm.at[idx_tok.at[j]], rows[b], sg[b])

        def wait_tok(j, b):
            pltpu.make_async_copy(tok_hbm.at[idx_tok.at[j]], rows[b],
                                  sg[b]).wait()

        def issue_add(j, b):
            pltpu.async_copy(comb_sp.at[idx_cmb.at[j]], rows[b], sa[b],
                             add=True)

        def wait_add(j, b):
            pltpu.make_async_copy(comb_sp.at[idx_cmb.at[j]], rows[b],
                                  sa[b]).wait()

        def issue_store(j, b):
            pltpu.async_copy(rows[b], out_at(j), so[b])

        def wait_store(j, b):
            pltpu.make_async_copy(rows[b], out_at(j), so[b]).wait()

        for j0 in range(LOOKAHEAD):
            issue_tok(j0, j0)

        pltpu.sync_copy(seg_hbm.at[wid], idx_cmb)

        tmp = rows[NBUF - 1]

        @pl.when(sid < MAX_LEN * NUM_SEG // PIECE)
        def _():
            pltpu.sync_copy(st_hbm, segv)
            poff = pl.multiple_of(lax.rem(sid, MAX_LEN // PIECE) * PIECE, 8)
            s2 = sid // (MAX_LEN // PIECE)
            pltpu.sync_copy(pos_hbm.at[pl.ds(poff, PIECE)],
                            tmp.at[pl.ds(0, PIECE)])

            def add_body(r, c):
                for g in range(EMBED // LANES):
                    sl = pl.ds(g * LANES, LANES)
                    tmp[r, sl] = tmp[r, sl] + segv[s2, sl]
                return c

            lax.fori_loop(0, PIECE, add_body, 0)
            coff = pl.multiple_of(sid * PIECE, 8)
            pltpu.sync_copy(tmp.at[pl.ds(0, PIECE)],
                            comb_sp.at[pl.ds(coff, PIECE)])

        def cidx_body(r, p):
            for g in range(EMBED // LANES):
                sl = pl.ds(g * LANES, LANES)
                idx_cmb[r, sl] = idx_cmb[r, sl] * MAX_LEN + p
                p = p + LANES
                p = jnp.where(p >= MAX_LEN, p - MAX_LEN, p)
            return p

        lax.fori_loop(0, CHUNKS_PER_W, cidx_body,
                      lax.broadcasted_iota(jnp.int32, (LANES,), 0))
        plsc.subcore_barrier()

        def phase(j, b, first=False, second=False, issue_next=True):
            wait_tok(j, b)
            issue_add(j, b)
            if not first:
                wait_add(j - 1, (b - 1) % NBUF)
                issue_store(j - 1, (b - 1) % NBUF)
            if not (first or second):
                wait_store(j - 2, (b - 2) % NBUF)
            if issue_next:
                issue_tok(j + LOOKAHEAD, (b + LOOKAHEAD) % NBUF)

        phase(0, 0, first=True)
        phase(1, 1, second=True)

        def body(i, carry):
            j0 = 2 + i * NBUF
            for t in range(NBUF):
                phase(j0 + t, (2 + t) % NBUF)
            return carry

        lax.fori_loop(0, (CHUNKS_PER_W - 2 - NBUF) // NBUF, body, 0)
        for j in range(CHUNKS_PER_W - NBUF, CHUNKS_PER_W):
            phase(j, j % NBUF,
                  issue_next=(j + LOOKAHEAD < CHUNKS_PER_W))
        j_last = CHUNKS_PER_W - 1
        wait_add(j_last, j_last % NBUF)
        issue_store(j_last, j_last % NBUF)
        wait_store(j_last - 1, (j_last - 1) % NBUF)
        wait_store(j_last, j_last % NBUF)

    return k(seq3d, seg3d, token_table, segmt_table, pos_table)


def kernel(sequence, segment, token_table, segmt_table, pos_table):
    seq = sequence.astype(jnp.int32).reshape(NW, CHUNKS_PER_W, CHUNK)
    seg = segment.astype(jnp.int32).reshape(NW, CHUNKS_PER_W, CHUNK)
    out = _sc_embed(seq, seg, token_table, segmt_table, pos_table)
    return out.reshape(BATCH, MAX_LEN, EMBED)
